# baseline (speedup 1.0000x reference)
import functools

import jax
import jax.numpy as jnp
from jax import lax
from jax.experimental import pallas as pl
from jax.experimental.pallas import tpu as pltpu
from jax.experimental.pallas import tpu_sc as plsc

B = 16384
D = 64
N_SPARSE = 8192
NC = 2
NS = 16
NW = NC * NS
ROWS_PER_W = B // NW
CHUNK = 128
NCHUNK = ROWS_PER_W // CHUNK
NVEC = N_SPARSE // 16


def _sc_gather_and_resolve(fixed_features, sf_index, sf_value, fixed_table,
                           sparse_table):
    mesh = plsc.VectorSubcoreMesh(core_axis_name="c", subcore_axis_name="s",
                                  num_cores=NC, num_subcores=NS)

    @functools.partial(
        pl.kernel,
        out_type=(
            jax.ShapeDtypeStruct((B, D), jnp.float32),
            jax.ShapeDtypeStruct((B, D), jnp.float32),
            jax.ShapeDtypeStruct((B,), jnp.int32),
        ),
        mesh=mesh,
        scratch_types=[
            pltpu.VMEM((NCHUNK, CHUNK), jnp.int32),
            pltpu.VMEM((N_SPARSE,), jnp.int32),
            pltpu.VMEM((N_SPARSE,), jnp.int32),
            pltpu.VMEM((NCHUNK, CHUNK), jnp.int32),
            pltpu.VMEM((NCHUNK, CHUNK), jnp.int32),
            pltpu.VMEM((32,), jnp.int32),
            pltpu.VMEM((ROWS_PER_W, D), jnp.float32),
            pltpu.VMEM((ROWS_PER_W, D), jnp.float32),
            pltpu.SemaphoreType.DMA,
        ],
        compiler_params=pltpu.CompilerParams(needs_layout_passes=False,
                                             use_tc_tiling_on_sc=False),
    )
    def k(ff_hbm, sfi_hbm, sfv_hbm, ftab_hbm, stab_hbm,
          fe_out, sr_out, sv_out,
          ffeat, sfi, sfv, sval, svc, scr, fe_rows, srows, sem):
        wid = lax.axis_index("c") * NS + lax.axis_index("s")
        base = wid * ROWS_PER_W

        for j in range(NCHUNK):
            pltpu.sync_copy(ff_hbm.at[pl.ds(base + CHUNK * j, CHUNK)],
                            ffeat.at[j])
        fcopies = [
            pltpu.async_copy(ftab_hbm.at[ffeat.at[j]],
                             fe_rows.at[pl.ds(CHUNK * j, CHUNK)], sem)
            for j in range(NCHUNK)
        ]

        pltpu.sync_copy(sfi_hbm, sfi)
        pltpu.sync_copy(sfv_hbm, sfv)

        neg1 = jnp.full((16,), -1, jnp.int32)
        for j in range(NCHUNK):
            for g in range(CHUNK // 16):
                sval[j, pl.ds(16 * g, 16)] = neg1
        scr[pl.ds(16, 16)] = neg1

        iota = lax.iota(jnp.int32, 16)
        iota1 = iota + 1

        UNROLL = 8

        def scan_body(u, carry):
            for s in range(UNROLL):
                v = u * UNROLL + s
                b16 = sfi[pl.ds(16 * v, 16)]
                key = (b16 << 13) | (iota + 16 * v)
                key_s = jnp.sort(key)
                scr[pl.ds(0, 16)] = key_s
                nxt = plsc.load_gather(scr, [iota1])
                b_s = key_s >> 13
                i_s = key_s & (N_SPARSE - 1)
                winner = (b_s != (nxt >> 13)) & ((b_s >> 9) == wid)
                sval_s = plsc.load_gather(sfv, [i_s])
                bl = b_s & (ROWS_PER_W - 1)
                plsc.store_scatter(sval, [bl >> 7, bl & (CHUNK - 1)],
                                   sval_s, mask=winner)
            return carry

        pass

        for j in range(NCHUNK):
            for g in range(CHUNK // 16):
                x = sval[j, pl.ds(16 * g, 16)]
                svc[j, pl.ds(16 * g, 16)] = jnp.maximum(x, 0)

        scopies = [
            pltpu.async_copy(stab_hbm.at[svc.at[j]],
                             srows.at[pl.ds(CHUNK * j, CHUNK)], sem)
            for j in range(NCHUNK)
        ]
        for c in fcopies + scopies:
            c.wait()

        pltpu.sync_copy(fe_rows, fe_out.at[pl.ds(base, ROWS_PER_W)])
        pltpu.sync_copy(srows, sr_out.at[pl.ds(base, ROWS_PER_W)])
        for j in range(NCHUNK):
            pltpu.sync_copy(sval.at[j],
                            sv_out.at[pl.ds(base + CHUNK * j, CHUNK)])

    return k(fixed_features, sf_index, sf_value, fixed_table, sparse_table)


BLK = 2048


def _mm_body(fe_ref, se_ref, sv_ref, w1_ref, w2_ref, b_ref, o_ref):
    mask = (sv_ref[...] >= 0).astype(jnp.float32)
    se = se_ref[...] * mask
    o_ref[...] = (
        jnp.dot(fe_ref[...], w1_ref[...], preferred_element_type=jnp.float32)
        + jnp.dot(se, w2_ref[...], preferred_element_type=jnp.float32)
        + b_ref[...]
    )


def _tc_matmul(fe, se, sv, W, b):
    grid = (B // BLK,)
    return pl.pallas_call(
        _mm_body,
        grid=grid,
        in_specs=[
            pl.BlockSpec((BLK, D), lambda i: (i, 0)),
            pl.BlockSpec((BLK, D), lambda i: (i, 0)),
            pl.BlockSpec((BLK, 1), lambda i: (i, 0)),
            pl.BlockSpec((D, D), lambda i: (0, 0)),
            pl.BlockSpec((D, D), lambda i: (0, 0)),
            pl.BlockSpec((1, D), lambda i: (0, 0)),
        ],
        out_specs=pl.BlockSpec((BLK, D), lambda i: (i, 0)),
        out_shape=jax.ShapeDtypeStruct((B, D), jnp.float32),
    )(fe, se, sv, W[:D], W[D:], b.reshape(1, D))


def kernel(fixed_features, sf_index, sf_value, fixed_table, sparse_table, W, b):
    fe, se, sv = _sc_gather_and_resolve(
        fixed_features.astype(jnp.int32),
        sf_index.astype(jnp.int32),
        sf_value.astype(jnp.int32),
        fixed_table, sparse_table)
    return _tc_matmul(fe, se, sv.reshape(B, 1), W, b)


# speedup vs baseline: 1.4330x; 1.4330x over previous
import functools

import jax
import jax.numpy as jnp
from jax import lax
from jax.experimental import pallas as pl
from jax.experimental.pallas import tpu as pltpu
from jax.experimental.pallas import tpu_sc as plsc

B = 16384
D = 64
N_SPARSE = 8192
NC = 2
NS = 16
NW = NC * NS
ROWS_PER_W = B // NW
CHUNK = 128
NCHUNK = ROWS_PER_W // CHUNK
NVEC = N_SPARSE // 16


def _sc_gather_and_resolve(fixed_features, sf_index, sf_value, fixed_table,
                           sparse_table):
    mesh = plsc.VectorSubcoreMesh(core_axis_name="c", subcore_axis_name="s",
                                  num_cores=NC, num_subcores=NS)

    @functools.partial(
        pl.kernel,
        out_type=(
            jax.ShapeDtypeStruct((B, D), jnp.float32),
            jax.ShapeDtypeStruct((B, D), jnp.float32),
        ),
        mesh=mesh,
        scratch_types=[
            pltpu.VMEM((NCHUNK, CHUNK), jnp.int32),
            pltpu.VMEM((N_SPARSE,), jnp.int32),
            pltpu.VMEM((N_SPARSE,), jnp.int32),
            pltpu.VMEM((NCHUNK, CHUNK), jnp.int32),
            pltpu.VMEM((NCHUNK, CHUNK), jnp.int32),
            pltpu.VMEM((NCHUNK, CHUNK), jnp.float32),
            pltpu.VMEM((32,), jnp.int32),
            pltpu.VMEM((ROWS_PER_W, D), jnp.float32),
            pltpu.VMEM((ROWS_PER_W, D), jnp.float32),
            pltpu.SemaphoreType.DMA,
        ],
        compiler_params=pltpu.CompilerParams(needs_layout_passes=False,
                                             use_tc_tiling_on_sc=False),
    )
    def k(ff_hbm, sfi_hbm, sfv_hbm, ftab_hbm, stab_hbm,
          fe_out, sr_out,
          ffeat, sfi, sfv, sval, svc, maskf, scr, fe_rows, srows, sem):
        wid = lax.axis_index("c") * NS + lax.axis_index("s")
        base = wid * ROWS_PER_W

        for j in range(NCHUNK):
            pltpu.sync_copy(ff_hbm.at[pl.ds(base + CHUNK * j, CHUNK)],
                            ffeat.at[j])
        fcopies = [
            pltpu.async_copy(ftab_hbm.at[ffeat.at[j]],
                             fe_rows.at[pl.ds(CHUNK * j, CHUNK)], sem)
            for j in range(NCHUNK)
        ]

        pltpu.sync_copy(sfi_hbm, sfi)
        pltpu.sync_copy(sfv_hbm, sfv)

        neg1 = jnp.full((16,), -1, jnp.int32)
        for j in range(NCHUNK):
            for g in range(CHUNK // 16):
                sval[j, pl.ds(16 * g, 16)] = neg1
        scr[pl.ds(16, 16)] = neg1

        iota = lax.iota(jnp.int32, 16)
        iota1 = iota + 1

        UNROLL = 8

        def scan_body(u, carry):
            for s in range(UNROLL):
                v = u * UNROLL + s
                b16 = sfi[pl.ds(16 * v, 16)]
                key = (b16 << 13) | (iota + 16 * v)
                key_s = jnp.sort(key)
                scr[pl.ds(0, 16)] = key_s
                nxt = plsc.load_gather(scr, [iota1])
                b_s = key_s >> 13
                i_s = key_s & (N_SPARSE - 1)
                winner = (b_s != (nxt >> 13)) & ((b_s >> 9) == wid)
                sval_s = plsc.load_gather(sfv, [i_s])
                bl = b_s & (ROWS_PER_W - 1)
                plsc.store_scatter(sval, [bl >> 7, bl & (CHUNK - 1)],
                                   sval_s, mask=winner)
            return carry

        lax.fori_loop(0, NVEC // UNROLL, scan_body, 0)

        for j in range(NCHUNK):
            for g in range(CHUNK // 16):
                x = sval[j, pl.ds(16 * g, 16)]
                slot = base + CHUNK * j + 16 * g + iota
                empty = x < 0
                svc[j, pl.ds(16 * g, 16)] = jnp.where(empty, slot, x)
                maskf[j, pl.ds(16 * g, 16)] = jnp.where(
                    empty, jnp.zeros((16,), jnp.float32),
                    jnp.ones((16,), jnp.float32))

        scopies = [
            pltpu.async_copy(stab_hbm.at[svc.at[j]],
                             srows.at[pl.ds(CHUNK * j, CHUNK)], sem)
            for j in range(NCHUNK)
        ]
        for c in fcopies:
            c.wait()
        pltpu.sync_copy(fe_rows, fe_out.at[pl.ds(base, ROWS_PER_W)])
        for c in scopies:
            c.wait()

        MUNROLL = 4

        def mask_body(u, carry):
            for s in range(MUNROLL):
                r = u * MUNROLL + s
                m16 = plsc.load_gather(
                    maskf, [jnp.full((16,), r, jnp.int32) >> 7,
                            jnp.full((16,), r, jnp.int32) & (CHUNK - 1)])
                row = srows.at[r]
                for c in range(D // 16):
                    row[pl.ds(16 * c, 16)] = row[pl.ds(16 * c, 16)] * m16
            return carry

        lax.fori_loop(0, ROWS_PER_W // MUNROLL, mask_body, 0)

        pltpu.sync_copy(srows, sr_out.at[pl.ds(base, ROWS_PER_W)])

    return k(fixed_features, sf_index, sf_value, fixed_table, sparse_table)


BLK = 2048


def _mm_body(fe_ref, se_ref, w1_ref, w2_ref, b_ref, o_ref):
    o_ref[...] = (
        jnp.dot(fe_ref[...], w1_ref[...], preferred_element_type=jnp.float32)
        + jnp.dot(se_ref[...], w2_ref[...], preferred_element_type=jnp.float32)
        + b_ref[...]
    )


def _tc_matmul(fe, se, W, b):
    grid = (B // BLK,)
    return pl.pallas_call(
        _mm_body,
        grid=grid,
        in_specs=[
            pl.BlockSpec((BLK, D), lambda i: (i, 0)),
            pl.BlockSpec((BLK, D), lambda i: (i, 0)),
            pl.BlockSpec((D, D), lambda i: (0, 0)),
            pl.BlockSpec((D, D), lambda i: (0, 0)),
            pl.BlockSpec((1, D), lambda i: (0, 0)),
        ],
        out_specs=pl.BlockSpec((BLK, D), lambda i: (i, 0)),
        out_shape=jax.ShapeDtypeStruct((B, D), jnp.float32),
    )(fe, se, W[:D], W[D:], b.reshape(1, D))


def kernel(fixed_features, sf_index, sf_value, fixed_table, sparse_table, W, b):
    fe, se = _sc_gather_and_resolve(
        fixed_features.astype(jnp.int32),
        sf_index.astype(jnp.int32),
        sf_value.astype(jnp.int32),
        fixed_table, sparse_table)
    return _tc_matmul(fe, se, W, b)


# speedup vs baseline: 1.5876x; 1.1078x over previous
import functools

import jax
import jax.numpy as jnp
from jax import lax
from jax.experimental import pallas as pl
from jax.experimental.pallas import tpu as pltpu
from jax.experimental.pallas import tpu_sc as plsc

B = 16384
D = 64
N_SPARSE = 8192
FIXED_V = 1000000
SPARSE_V = 100000
NC = 2
NS = 16
NW = NC * NS
ROWS_PER_W = B // NW
CHUNK = 128
NCHUNK = ROWS_PER_W // CHUNK
NVEC = N_SPARSE // 16


def _sc_gather_and_resolve(fixed_features, sf_index, sf_value, ftp, stp):
    mesh = plsc.VectorSubcoreMesh(core_axis_name="c", subcore_axis_name="s",
                                  num_cores=NC, num_subcores=NS)

    @functools.partial(
        pl.kernel,
        out_type=(
            jax.ShapeDtypeStruct((B, 2 * D), jnp.float32),
            jax.ShapeDtypeStruct((B, 2 * D), jnp.float32),
        ),
        mesh=mesh,
        scratch_types=[
            pltpu.VMEM((NCHUNK, CHUNK), jnp.int32),
            pltpu.VMEM((N_SPARSE,), jnp.int32),
            pltpu.VMEM((N_SPARSE,), jnp.int32),
            pltpu.VMEM((NCHUNK, CHUNK), jnp.int32),
            pltpu.VMEM((NCHUNK, CHUNK), jnp.int32),
            pltpu.VMEM((32,), jnp.int32),
            pltpu.VMEM((ROWS_PER_W, 2 * D), jnp.float32),
            pltpu.SemaphoreType.DMA,
        ],
        compiler_params=pltpu.CompilerParams(needs_layout_passes=False,
                                             use_tc_tiling_on_sc=False),
    )
    def k(ff_hbm, sfi_hbm, sfv_hbm, ftp_hbm, stp_hbm,
          fe_out, sr_out,
          ffeat, sfi, sfv, sval, svc, scr, rows, sem):
        wid = lax.axis_index("c") * NS + lax.axis_index("s")
        base = wid * ROWS_PER_W
        iota = lax.iota(jnp.int32, 16)
        iota1 = iota + 1
        zero16 = jnp.zeros((16,), jnp.float32)

        for j in range(NCHUNK):
            pltpu.sync_copy(ff_hbm.at[pl.ds(base + CHUNK * j, CHUNK)],
                            ffeat.at[j])
        fcopies = [
            pltpu.async_copy(ftp_hbm.at[ffeat.at[j]],
                             rows.at[pl.ds(CHUNK * j, CHUNK)], sem)
            for j in range(NCHUNK)
        ]

        pltpu.sync_copy(sfi_hbm, sfi)
        pltpu.sync_copy(sfv_hbm, sfv)

        neg1 = jnp.full((16,), -1, jnp.int32)
        for j in range(NCHUNK):
            for g in range(CHUNK // 16):
                sval[j, pl.ds(16 * g, 16)] = neg1
        scr[pl.ds(16, 16)] = neg1

        UNROLL = 8

        def scan_body(u, carry):
            for s in range(UNROLL):
                v = u * UNROLL + s
                b16 = sfi[pl.ds(16 * v, 16)]
                key = (b16 << 13) | (iota + 16 * v)
                key_s = jnp.sort(key)
                scr[pl.ds(0, 16)] = key_s
                nxt = plsc.load_gather(scr, [iota1])
                b_s = key_s >> 13
                i_s = key_s & (N_SPARSE - 1)
                winner = (b_s != (nxt >> 13)) & ((b_s >> 9) == wid)
                sval_s = plsc.load_gather(sfv, [i_s])
                bl = b_s & (ROWS_PER_W - 1)
                plsc.store_scatter(sval, [bl >> 7, bl & (CHUNK - 1)],
                                   sval_s, mask=winner)
            return carry

        lax.fori_loop(0, NVEC // UNROLL, scan_body, 0)

        for j in range(NCHUNK):
            for g in range(CHUNK // 16):
                x = sval[j, pl.ds(16 * g, 16)]
                slot = base + CHUNK * j + 16 * g + iota
                svc[j, pl.ds(16 * g, 16)] = jnp.where(x < 0, slot, x)

        for j in range(NCHUNK):
            fcopies[j].wait()
            pltpu.sync_copy(rows.at[pl.ds(CHUNK * j, CHUNK)],
                            fe_out.at[pl.ds(base + CHUNK * j, CHUNK)])

        scopies = [
            pltpu.async_copy(stp_hbm.at[svc.at[j]],
                             rows.at[pl.ds(CHUNK * j, CHUNK)], sem)
            for j in range(NCHUNK)
        ]
        for j in range(NCHUNK):
            scopies[j].wait()

            def szero_body(g, carry, j=j):
                v16 = sval[j, pl.ds(16 * g, 16)]
                r16 = CHUNK * j + 16 * g + iota
                empty = v16 < 0
                col0 = iota & 0
                for c in range(D):
                    plsc.store_scatter(rows, [r16, col0 + c], zero16,
                                       mask=empty)
                return carry

            lax.fori_loop(0, CHUNK // 16, szero_body, 0)
            pltpu.sync_copy(rows.at[pl.ds(CHUNK * j, CHUNK)],
                            sr_out.at[pl.ds(base + CHUNK * j, CHUNK)])

    return k(fixed_features, sf_index, sf_value, ftp, stp)


BLK = 2048


def _mm_body(fe_ref, se_ref, w1_ref, w2_ref, b_ref, o_ref):
    o_ref[...] = (
        jnp.dot(fe_ref[...], w1_ref[...], preferred_element_type=jnp.float32)
        + jnp.dot(se_ref[...], w2_ref[...], preferred_element_type=jnp.float32)
        + b_ref[...]
    )


def _tc_matmul(fe, se, W, b):
    zpad = jnp.zeros((D, D), jnp.float32)
    W1p = jnp.concatenate([W[:D], zpad], axis=0)
    W2p = jnp.concatenate([W[D:], zpad], axis=0)
    grid = (B // BLK,)
    return pl.pallas_call(
        _mm_body,
        grid=grid,
        in_specs=[
            pl.BlockSpec((BLK, 2 * D), lambda i: (i, 0)),
            pl.BlockSpec((BLK, 2 * D), lambda i: (i, 0)),
            pl.BlockSpec((2 * D, D), lambda i: (0, 0)),
            pl.BlockSpec((2 * D, D), lambda i: (0, 0)),
            pl.BlockSpec((1, D), lambda i: (0, 0)),
        ],
        out_specs=pl.BlockSpec((BLK, D), lambda i: (i, 0)),
        out_shape=jax.ShapeDtypeStruct((B, D), jnp.float32),
    )(fe, se, W1p, W2p, b.reshape(1, D))


def kernel(fixed_features, sf_index, sf_value, fixed_table, sparse_table, W, b):
    ftp = jnp.pad(fixed_table, ((0, 0), (0, D)))
    stp = jnp.pad(sparse_table, ((0, 0), (0, D)))
    fe, se = _sc_gather_and_resolve(
        fixed_features.astype(jnp.int32),
        sf_index.astype(jnp.int32),
        sf_value.astype(jnp.int32),
        ftp, stp)
    return _tc_matmul(fe, se, W, b)


# speedup vs baseline: 2.4183x; 1.5233x over previous
import functools

import jax
import jax.numpy as jnp
from jax import lax
from jax.experimental import pallas as pl
from jax.experimental.pallas import tpu as pltpu
from jax.experimental.pallas import tpu_sc as plsc

B = 16384
D = 64
N_SPARSE = 8192
NC = 2
NS = 16
NW = NC * NS
ROWS_PER_W = B // NW
NVEC = N_SPARSE // 16


def _mesh():
    return plsc.VectorSubcoreMesh(core_axis_name="c", subcore_axis_name="s",
                                  num_cores=NC, num_subcores=NS)


def _sc_scan(sf_index, sf_value):
    @functools.partial(
        pl.kernel,
        out_type=jax.ShapeDtypeStruct((B,), jnp.int32),
        mesh=_mesh(),
        scratch_types=[
            pltpu.VMEM((N_SPARSE,), jnp.int32),
            pltpu.VMEM((N_SPARSE,), jnp.int32),
            pltpu.VMEM((4, 128), jnp.int32),
            pltpu.VMEM((32,), jnp.int32),
        ],
        compiler_params=pltpu.CompilerParams(needs_layout_passes=False,
                                             use_tc_tiling_on_sc=False),
    )
    def k(sfi_hbm, sfv_hbm, sv_out, sfi, sfv, sval, scr):
        wid = lax.axis_index("c") * NS + lax.axis_index("s")
        iota = lax.iota(jnp.int32, 16)
        iota1 = iota + 1

        pltpu.sync_copy(sfi_hbm, sfi)
        pltpu.sync_copy(sfv_hbm, sfv)

        neg1 = jnp.full((16,), -1, jnp.int32)
        for j in range(4):
            for g in range(8):
                sval[j, pl.ds(16 * g, 16)] = neg1
        scr[pl.ds(16, 16)] = neg1

        UNROLL = 8

        def scan_body(u, carry):
            for s in range(UNROLL):
                v = u * UNROLL + s
                b16 = sfi[pl.ds(16 * v, 16)]
                key = (b16 << 13) | (iota + 16 * v)
                key_s = jnp.sort(key)
                scr[pl.ds(0, 16)] = key_s
                nxt = plsc.load_gather(scr, [iota1])
                b_s = key_s >> 13
                i_s = key_s & (N_SPARSE - 1)
                winner = (b_s != (nxt >> 13)) & ((b_s >> 9) == wid)
                sval_s = plsc.load_gather(sfv, [i_s])
                bl = b_s & (ROWS_PER_W - 1)
                plsc.store_scatter(sval, [bl >> 7, bl & 127],
                                   sval_s, mask=winner)
            return carry

        lax.fori_loop(0, NVEC // UNROLL, scan_body, 0)

        for j in range(4):
            pltpu.sync_copy(sval.at[j],
                            sv_out.at[pl.ds(wid * ROWS_PER_W + 128 * j, 128)])

    return k(sf_index, sf_value)


def _sc_row_fetch(idx, table, skip_negative):
    @functools.partial(
        pl.kernel,
        out_type=jax.ShapeDtypeStruct((B, D), jnp.float32),
        mesh=_mesh(),
        scratch_types=[
            pltpu.VMEM((ROWS_PER_W,), jnp.int32),
            pltpu.VMEM((ROWS_PER_W, D), jnp.float32),
            pltpu.SemaphoreType.DMA,
        ],
        compiler_params=pltpu.CompilerParams(needs_layout_passes=False,
                                             use_tc_tiling_on_sc=True),
    )
    def k(idx_hbm, tab_hbm, out_hbm, idx_v, rows, sem):
        wid = lax.axis_index("c") * NS + lax.axis_index("s")
        base = wid * ROWS_PER_W

        pltpu.sync_copy(idx_hbm.at[pl.ds(base, ROWS_PER_W)], idx_v)

        if skip_negative:
            zero16 = jnp.zeros((16,), jnp.float32)

            def zero_body(r, carry):
                for c in range(D // 16):
                    rows[r, pl.ds(16 * c, 16)] = zero16
                return carry

            lax.fori_loop(0, ROWS_PER_W, zero_body, 0)

        def fire_body(g, carry):
            v16 = idx_v[pl.ds(16 * g, 16)]
            for l in range(16):
                v = v16[l]
                r = 16 * g + l
                if skip_negative:
                    @pl.when(v >= 0)
                    def _(v=v, r=r):
                        pltpu.async_copy(tab_hbm.at[pl.ds(v, 1)],
                                         rows.at[pl.ds(r, 1)], sem)
                else:
                    pltpu.async_copy(tab_hbm.at[pl.ds(v, 1)],
                                     rows.at[pl.ds(r, 1)], sem)
            return carry

        lax.fori_loop(0, ROWS_PER_W // 16, fire_body, 0)

        if skip_negative:
            def drain_body(g, carry):
                v16 = idx_v[pl.ds(16 * g, 16)]
                for l in range(16):
                    v = v16[l]
                    r = 16 * g + l

                    @pl.when(v >= 0)
                    def _(r=r):
                        pltpu.make_async_copy(tab_hbm.at[pl.ds(0, 1)],
                                              rows.at[pl.ds(r, 1)],
                                              sem).wait()
                return carry

            lax.fori_loop(0, ROWS_PER_W // 16, drain_body, 0)
        else:
            pltpu.make_async_copy(tab_hbm.at[pl.ds(0, ROWS_PER_W)],
                                  rows, sem).wait()

        pltpu.sync_copy(rows, out_hbm.at[pl.ds(base, ROWS_PER_W)])

    return k(idx, table)


BLK = 2048


def _mm_body(fe_ref, se_ref, w1_ref, w2_ref, b_ref, o_ref):
    o_ref[...] = (
        jnp.dot(fe_ref[...], w1_ref[...], preferred_element_type=jnp.float32)
        + jnp.dot(se_ref[...], w2_ref[...], preferred_element_type=jnp.float32)
        + b_ref[...]
    )


def _tc_matmul(fe, se, W, b):
    grid = (B // BLK,)
    return pl.pallas_call(
        _mm_body,
        grid=grid,
        in_specs=[
            pl.BlockSpec((BLK, D), lambda i: (i, 0)),
            pl.BlockSpec((BLK, D), lambda i: (i, 0)),
            pl.BlockSpec((D, D), lambda i: (0, 0)),
            pl.BlockSpec((D, D), lambda i: (0, 0)),
            pl.BlockSpec((1, D), lambda i: (0, 0)),
        ],
        out_specs=pl.BlockSpec((BLK, D), lambda i: (i, 0)),
        out_shape=jax.ShapeDtypeStruct((B, D), jnp.float32),
    )(fe, se, W[:D], W[D:], b.reshape(1, D))


def kernel(fixed_features, sf_index, sf_value, fixed_table, sparse_table, W, b):
    sv = _sc_scan(sf_index.astype(jnp.int32), sf_value.astype(jnp.int32))
    fe = _sc_row_fetch(fixed_features.astype(jnp.int32), fixed_table,
                       skip_negative=False)
    se = _sc_row_fetch(sv, sparse_table, skip_negative=True)
    return _tc_matmul(fe, se, W, b)


# speedup vs baseline: 2.4538x; 1.0147x over previous
import functools

import jax
import jax.numpy as jnp
from jax import lax
from jax.experimental import pallas as pl
from jax.experimental.pallas import tpu as pltpu
from jax.experimental.pallas import tpu_sc as plsc

B = 16384
D = 64
N_SPARSE = 8192
NC = 2
NS = 16
NW = NC * NS
ROWS_PER_W = B // NW
NVEC = N_SPARSE // 16


def _mesh():
    return plsc.VectorSubcoreMesh(core_axis_name="c", subcore_axis_name="s",
                                  num_cores=NC, num_subcores=NS)


def _sc_scan(sf_index, sf_value):
    @functools.partial(
        pl.kernel,
        out_type=jax.ShapeDtypeStruct((B,), jnp.int32),
        mesh=_mesh(),
        scratch_types=[
            pltpu.VMEM((N_SPARSE,), jnp.int32),
            pltpu.VMEM((N_SPARSE,), jnp.int32),
            pltpu.VMEM((4, 128), jnp.int32),
            pltpu.VMEM((32,), jnp.int32),
        ],
        compiler_params=pltpu.CompilerParams(needs_layout_passes=False,
                                             use_tc_tiling_on_sc=False),
    )
    def k(sfi_hbm, sfv_hbm, sv_out, sfi, sfv, sval, scr):
        wid = lax.axis_index("c") * NS + lax.axis_index("s")
        iota = lax.iota(jnp.int32, 16)
        iota1 = iota + 1

        pltpu.sync_copy(sfi_hbm, sfi)
        pltpu.sync_copy(sfv_hbm, sfv)

        neg1 = jnp.full((16,), -1, jnp.int32)
        for j in range(4):
            for g in range(8):
                sval[j, pl.ds(16 * g, 16)] = neg1
        scr[pl.ds(16, 16)] = neg1

        UNROLL = 8

        def scan_body(u, carry):
            for s in range(UNROLL):
                v = u * UNROLL + s
                b16 = sfi[pl.ds(16 * v, 16)]
                key = (b16 << 13) | (iota + 16 * v)
                key_s = jnp.sort(key)
                scr[pl.ds(0, 16)] = key_s
                nxt = plsc.load_gather(scr, [iota1])
                b_s = key_s >> 13
                i_s = key_s & (N_SPARSE - 1)
                winner = (b_s != (nxt >> 13)) & ((b_s >> 9) == wid)
                sval_s = plsc.load_gather(sfv, [i_s])
                bl = b_s & (ROWS_PER_W - 1)
                plsc.store_scatter(sval, [bl >> 7, bl & 127],
                                   sval_s, mask=winner)
            return carry

        lax.fori_loop(0, NVEC // UNROLL, scan_body, 0)

        for j in range(4):
            pltpu.sync_copy(sval.at[j],
                            sv_out.at[pl.ds(wid * ROWS_PER_W + 128 * j, 128)])

    return k(sf_index, sf_value)


def _sc_row_fetch(idx, table, skip_negative):
    @functools.partial(
        pl.kernel,
        out_type=jax.ShapeDtypeStruct((B, D), jnp.float32),
        mesh=_mesh(),
        scratch_types=[
            pltpu.VMEM((ROWS_PER_W,), jnp.int32),
            pltpu.VMEM((ROWS_PER_W, D), jnp.float32),
            pltpu.SemaphoreType.DMA,
        ],
        compiler_params=pltpu.CompilerParams(needs_layout_passes=False,
                                             use_tc_tiling_on_sc=True),
    )
    def k(idx_hbm, tab_hbm, out_hbm, idx_v, rows, sem):
        wid = lax.axis_index("c") * NS + lax.axis_index("s")
        base = wid * ROWS_PER_W

        pltpu.sync_copy(idx_hbm.at[pl.ds(base, ROWS_PER_W)], idx_v)

        if skip_negative:
            zero16 = jnp.zeros((16,), jnp.float32)

            def zero_body(r, carry):
                for c in range(D // 16):
                    rows[r, pl.ds(16 * c, 16)] = zero16
                return carry

            lax.fori_loop(0, ROWS_PER_W, zero_body, 0)

        def fire_body(g, carry):
            v16 = idx_v[pl.ds(16 * g, 16)]
            for l in range(16):
                v = v16[l]
                r = 16 * g + l
                if skip_negative:
                    @pl.when(v >= 0)
                    def _(v=v, r=r):
                        pltpu.async_copy(tab_hbm.at[pl.ds(v, 1)],
                                         rows.at[pl.ds(r, 1)], sem)
                else:
                    pltpu.async_copy(tab_hbm.at[pl.ds(v, 1)],
                                     rows.at[pl.ds(r, 1)], sem)
            return carry

        lax.fori_loop(0, ROWS_PER_W // 16, fire_body, 0)

        if skip_negative:
            def drain_body(g, carry):
                v16 = idx_v[pl.ds(16 * g, 16)]
                for l in range(16):
                    v = v16[l]
                    r = 16 * g + l

                    @pl.when(v >= 0)
                    def _(r=r):
                        pltpu.make_async_copy(tab_hbm.at[pl.ds(0, 1)],
                                              rows.at[pl.ds(r, 1)],
                                              sem).wait()
                return carry

            lax.fori_loop(0, ROWS_PER_W // 16, drain_body, 0)
        else:
            pltpu.make_async_copy(tab_hbm.at[pl.ds(0, ROWS_PER_W)],
                                  rows, sem).wait()

        pltpu.sync_copy(rows, out_hbm.at[pl.ds(base, ROWS_PER_W)])

    return k(idx, table)


BLK = 2048


def _mm_body(fe_ref, se_ref, w1_ref, w2_ref, b_ref, o_ref):
    acc = lax.dot_general(w1_ref[...], fe_ref[...], (((0,), (1,)), ((), ())),
                          preferred_element_type=jnp.float32)
    acc = acc + lax.dot_general(w2_ref[...], se_ref[...],
                                (((0,), (1,)), ((), ())),
                                preferred_element_type=jnp.float32)
    o_ref[...] = acc + b_ref[...]


def _tc_matmul(fe, se, W, b):
    grid = (B // BLK,)
    outT = pl.pallas_call(
        _mm_body,
        grid=grid,
        in_specs=[
            pl.BlockSpec((BLK, D), lambda i: (i, 0)),
            pl.BlockSpec((BLK, D), lambda i: (i, 0)),
            pl.BlockSpec((D, D), lambda i: (0, 0)),
            pl.BlockSpec((D, D), lambda i: (0, 0)),
            pl.BlockSpec((D, 1), lambda i: (0, 0)),
        ],
        out_specs=pl.BlockSpec((D, BLK), lambda i: (0, i)),
        out_shape=jax.ShapeDtypeStruct((D, B), jnp.float32),
    )(fe, se, W[:D], W[D:], b.reshape(D, 1))
    return outT.T


def kernel(fixed_features, sf_index, sf_value, fixed_table, sparse_table, W, b):
    sv = _sc_scan(sf_index.astype(jnp.int32), sf_value.astype(jnp.int32))
    fe = _sc_row_fetch(fixed_features.astype(jnp.int32), fixed_table,
                       skip_negative=False)
    se = _sc_row_fetch(sv, sparse_table, skip_negative=True)
    return _tc_matmul(fe, se, W, b)
